# baseline (speedup 1.0000x reference)
import math

import jax
import jax.numpy as jnp
from jax.experimental import pallas as pl
from jax.experimental.pallas import tpu as pltpu

_L = 512
_NK = 25
_ROWS = 16
_TWO_PI = 2.0 * math.pi


def _cross(a, b):
    ax, ay, az = a
    bx, by, bz = b
    return (ay * bz - az * by, az * bx - ax * bz, ax * by - ay * bx)


def _body(cut_ref, ni_ref, cai_ref, cbi_ref, cbj_ref, coeff_ref, mask_ref,
          out_ref):
    step_idx = pl.program_id(0)
    c0 = cut_ref[0, 0]
    h = cut_ref[0, 1] - cut_ref[0, 0]
    rh = 1.0 / h
    h2_6 = h * h * (1.0 / 6.0)

    n = ni_ref[...]
    ca = cai_ref[...]
    cb = cbi_ref[...]
    nc = (n[0], n[1], n[2])
    cac = (ca[0], ca[1], ca[2])
    cbc = (cb[0], cb[1], cb[2])
    b0 = tuple(cac[k] - nc[k] for k in range(3))
    b1 = tuple(cbc[k] - cac[k] for k in range(3))
    n1 = _cross(b0, b1)
    A = _cross(n1, b1)
    nrm = jnp.sqrt(b1[0] * b1[0] + b1[1] * b1[1] + b1[2] * b1[2]) + 1e-9
    Braw = _cross(A, b1)
    Bv = tuple(Braw[k] / nrm for k in range(3))
    cx = -(A[0] * cbc[0] + A[1] * cbc[1] + A[2] * cbc[2])
    cy = -(Bv[0] * cbc[0] + Bv[1] * cbc[1] + Bv[2] * cbc[2])

    cbj = cbj_ref[...]
    mf = mask_ref[...]

    nb = ni_ref.shape[2]
    qs = []
    stus = []
    for b in range(nb):
        ax = A[0][:, b:b + 1]
        ay = A[1][:, b:b + 1]
        az = A[2][:, b:b + 1]
        bx = Bv[0][:, b:b + 1]
        by = Bv[1][:, b:b + 1]
        bz = Bv[2][:, b:b + 1]
        jx = cbj[0, b][None, :]
        jy = cbj[1, b][None, :]
        jz = cbj[2, b][None, :]
        X = ax * jx + ay * jy + az * jz + cx[:, b:b + 1]
        Y = bx * jx + by * jy + bz * jz + cy[:, b:b + 1]
        theta = jnp.arctan2(Y, X)
        q = (jnp.where(theta < c0, theta + _TWO_PI, theta) - c0) * rh
        u = q - jnp.floor(q)
        qs.append(q)
        stus.append((u * u - u) * h2_6)

    acc = jnp.zeros(mf.shape, jnp.float32)
    for k in range(_NK):
        cyk = coeff_ref[k].astype(jnp.float32)
        cmk = coeff_ref[k + _NK].astype(jnp.float32)
        for b in range(nb):
            g = 1.0 - jnp.abs(qs[b] - float(k))
            hit = g > 0.0
            p = jnp.where(hit, g, 0.0)
            w2 = jnp.where(hit, (g + 1.0) * stus[b], 0.0)
            acc = acc + p * cyk + w2 * cmk

    partial = jnp.sum(acc * mf)[None, None]

    @pl.when(step_idx == 0)
    def _():
        out_ref[...] = jnp.zeros((1, 1), jnp.float32)

    out_ref[...] += partial


_NSPLIT = 2


def kernel(N, CA, CB, coeff, cutoffs, mask):
    L = mask.shape[0]
    nb = N.shape[0]
    ni = jnp.transpose(N, (2, 1, 0))
    cai = jnp.transpose(CA, (2, 1, 0))
    cbi = jnp.transpose(CB, (2, 1, 0))
    cbj = jnp.transpose(CB, (2, 0, 1))
    mf = mask.astype(jnp.float32)
    cuts = cutoffs.reshape(1, _NK)

    rows = L // _NSPLIT
    total = None
    for s in range(_NSPLIT):
        sl = slice(s * rows, (s + 1) * rows)
        c2 = jnp.transpose(
            coeff[sl].reshape(rows, L, 2 * _NK).astype(jnp.bfloat16),
            (2, 0, 1))
        out = pl.pallas_call(
            _body,
            grid=(rows // _ROWS,),
            in_specs=[
                pl.BlockSpec(memory_space=pltpu.SMEM),
                pl.BlockSpec((3, _ROWS, nb), lambda i: (0, i, 0)),
                pl.BlockSpec((3, _ROWS, nb), lambda i: (0, i, 0)),
                pl.BlockSpec((3, _ROWS, nb), lambda i: (0, i, 0)),
                pl.BlockSpec((3, nb, L), lambda i: (0, 0, 0)),
                pl.BlockSpec((2 * _NK, _ROWS, L), lambda i: (0, i, 0)),
                pl.BlockSpec((_ROWS, L), lambda i: (i, 0)),
            ],
            out_specs=pl.BlockSpec((1, 1), lambda i: (0, 0)),
            out_shape=jax.ShapeDtypeStruct((1, 1), jnp.float32),
            compiler_params=pltpu.CompilerParams(
                dimension_semantics=("arbitrary",)),
        )(cuts, ni[:, sl], cai[:, sl], cbi[:, sl], cbj, c2, mf[sl])
        total = out[0, 0] if total is None else total + out[0, 0]
    return total


# speedup vs baseline: 1.0889x; 1.0889x over previous
import math

import jax
import jax.numpy as jnp
from jax.experimental import pallas as pl
from jax.experimental.pallas import tpu as pltpu

_L = 512
_NK = 25
_ROWS = 16
_TWO_PI = 2.0 * math.pi


def _cross(a, b):
    ax, ay, az = a
    bx, by, bz = b
    return (ay * bz - az * by, az * bx - ax * bz, ax * by - ay * bx)


def _body(cut_ref, ni_ref, cai_ref, cbi_ref, cbj_ref, coeff_ref, mask_ref,
          out_ref):
    step_idx = pl.program_id(0)
    c0 = cut_ref[0, 0]
    h = cut_ref[0, 1] - cut_ref[0, 0]
    rh = 1.0 / h
    h2_6 = h * h * (1.0 / 6.0)

    n = ni_ref[...]
    ca = cai_ref[...]
    cb = cbi_ref[...]
    nc = (n[0], n[1], n[2])
    cac = (ca[0], ca[1], ca[2])
    cbc = (cb[0], cb[1], cb[2])
    b0 = tuple(cac[k] - nc[k] for k in range(3))
    b1 = tuple(cbc[k] - cac[k] for k in range(3))
    n1 = _cross(b0, b1)
    A = _cross(n1, b1)
    nrm = jnp.sqrt(b1[0] * b1[0] + b1[1] * b1[1] + b1[2] * b1[2]) + 1e-9
    Braw = _cross(A, b1)
    Bv = tuple(Braw[k] / nrm for k in range(3))
    cx = -(A[0] * cbc[0] + A[1] * cbc[1] + A[2] * cbc[2])
    cy = -(Bv[0] * cbc[0] + Bv[1] * cbc[1] + Bv[2] * cbc[2])

    cbj = cbj_ref[...]
    mf = mask_ref[...]

    nb = ni_ref.shape[2]
    qs = []
    stus = []
    for b in range(nb):
        ax = A[0][:, b:b + 1]
        ay = A[1][:, b:b + 1]
        az = A[2][:, b:b + 1]
        bx = Bv[0][:, b:b + 1]
        by = Bv[1][:, b:b + 1]
        bz = Bv[2][:, b:b + 1]
        jx = cbj[0, b][None, :]
        jy = cbj[1, b][None, :]
        jz = cbj[2, b][None, :]
        X = ax * jx + ay * jy + az * jz + cx[:, b:b + 1]
        Y = bx * jx + by * jy + bz * jz + cy[:, b:b + 1]
        theta = jnp.arctan2(Y, X)
        q = (jnp.where(theta < c0, theta + _TWO_PI, theta) - c0) * rh
        u = q - jnp.floor(q)
        qs.append(q)
        stus.append((u * u - u) * h2_6)

    acc = jnp.zeros(mf.shape, jnp.float32)
    for k in range(_NK):
        cyk = coeff_ref[k].astype(jnp.float32)
        cmk = coeff_ref[k + _NK].astype(jnp.float32)
        for b in range(nb):
            g = 1.0 - jnp.abs(qs[b] - float(k))
            hit = g > 0.0
            p = jnp.where(hit, g, 0.0)
            w2 = jnp.where(hit, (g + 1.0) * stus[b], 0.0)
            acc = acc + p * cyk + w2 * cmk

    partial = jnp.sum(acc * mf)[None, None]

    @pl.when(step_idx == 0)
    def _():
        out_ref[...] = jnp.zeros((1, 1), jnp.float32)

    out_ref[...] += partial


_NSPLIT = 1


def kernel(N, CA, CB, coeff, cutoffs, mask):
    L = mask.shape[0]
    nb = N.shape[0]
    ni = jnp.transpose(N, (2, 1, 0))
    cai = jnp.transpose(CA, (2, 1, 0))
    cbi = jnp.transpose(CB, (2, 1, 0))
    cbj = jnp.transpose(CB, (2, 0, 1))
    mf = mask.astype(jnp.float32)
    cuts = cutoffs.reshape(1, _NK)

    rows = L // _NSPLIT
    total = None
    for s in range(_NSPLIT):
        sl = slice(s * rows, (s + 1) * rows)
        c2 = jnp.transpose(
            coeff[sl].reshape(rows, L, 2 * _NK).astype(jnp.bfloat16),
            (2, 0, 1))
        out = pl.pallas_call(
            _body,
            grid=(rows // _ROWS,),
            in_specs=[
                pl.BlockSpec(memory_space=pltpu.SMEM),
                pl.BlockSpec((3, _ROWS, nb), lambda i: (0, i, 0)),
                pl.BlockSpec((3, _ROWS, nb), lambda i: (0, i, 0)),
                pl.BlockSpec((3, _ROWS, nb), lambda i: (0, i, 0)),
                pl.BlockSpec((3, nb, L), lambda i: (0, 0, 0)),
                pl.BlockSpec((2 * _NK, _ROWS, L), lambda i: (0, i, 0)),
                pl.BlockSpec((_ROWS, L), lambda i: (i, 0)),
            ],
            out_specs=pl.BlockSpec((1, 1), lambda i: (0, 0)),
            out_shape=jax.ShapeDtypeStruct((1, 1), jnp.float32),
            compiler_params=pltpu.CompilerParams(
                dimension_semantics=("arbitrary",)),
        )(cuts, ni[:, sl], cai[:, sl], cbi[:, sl], cbj, c2, mf[sl])
        total = out[0, 0] if total is None else total + out[0, 0]
    return total
